# Initial kernel scaffold; baseline (speedup 1.0000x reference)
import jax
import jax.numpy as jnp
from jax.experimental import pallas as pl


def kernel(vertices, params):
    raise NotImplementedError("write your pallas kernel here")


# speedup vs baseline: 3.6786x; 3.6786x over previous
import jax
import jax.numpy as jnp
from jax import lax
from jax.experimental import pallas as pl


def _knn_body(verts_ref, ni_ref, dn_ref):
    x = verts_ref[0]
    V = x.shape[0]
    inner = lax.dot_general(x, x, (((1,), (1,)), ((), ())),
                            preferred_element_type=jnp.float32)
    qcol = jnp.sum(x * x, axis=1, keepdims=True)
    row_i = lax.broadcasted_iota(jnp.int32, (V, V), 0)
    col_i = lax.broadcasted_iota(jnp.int32, (V, V), 1)
    eye = row_i == col_i
    qrow = jnp.sum(jnp.where(eye, qcol, 0.0), axis=0, keepdims=True)
    D = -2.0 * inner + qcol + qrow
    xr = [jnp.sum(jnp.where(eye, x[:, c:c + 1], 0.0), axis=0, keepdims=True)
          for c in range(3)]
    for k in range(21):
        mv = jnp.min(D, axis=1, keepdims=True)
        idx = jnp.min(jnp.where(D == mv, col_i, V), axis=1, keepdims=True)
        m = col_i == idx
        D = jnp.where(m, 1e30, D)
        if k > 0:
            ni_ref[0, :, k - 1:k] = idx
            nv = [jnp.sum(jnp.where(m, xr[c], 0.0), axis=1, keepdims=True)
                  for c in range(3)]
            dv = [nv[c] - x[:, c:c + 1] for c in range(3)]
            nrm = jnp.sqrt(dv[0] * dv[0] + dv[1] * dv[1] + dv[2] * dv[2]) + 1e-12
            dn_ref[0, :, 3 * (k - 1):3 * k] = jnp.concatenate(
                [dv[c] / nrm for c in range(3)], axis=1)


def _knn(verts):
    bs, V, _ = verts.shape
    return pl.pallas_call(
        _knn_body,
        grid=(bs,),
        in_specs=[pl.BlockSpec((1, V, 3), lambda b: (b, 0, 0))],
        out_specs=(pl.BlockSpec((1, V, 20), lambda b: (b, 0, 0)),
                   pl.BlockSpec((1, V, 60), lambda b: (b, 0, 0))),
        out_shape=(jax.ShapeDtypeStruct((bs, V, 20), jnp.int32),
                   jax.ShapeDtypeStruct((bs, V, 60), jnp.float32)),
    )(verts)


def _matmul_body(fm_ref, wc_ref, ws_ref, bc_ref, bs_ref, fc_ref, fs_ref):
    x = fm_ref[0]
    fc_ref[0] = jnp.dot(x, wc_ref[...],
                        preferred_element_type=jnp.float32) + bc_ref[...]
    fs_ref[0] = jnp.dot(x, ws_ref[...],
                        preferred_element_type=jnp.float32) + bs_ref[...]


def _agg_body(fc_ref, fs_ref, ni_ref, dn_ref, d_ref, out_ref):
    fs = fs_ref[0]
    V = fs.shape[0]
    d = d_ref[...]
    nrm = jnp.sqrt(jnp.sum(d * d, axis=0, keepdims=True))
    s = d / (nrm + 1e-12)
    sb = s.astype(jnp.bfloat16).astype(jnp.float32)
    s0, s1, s2 = sb[0:1, :], sb[1:2, :], sb[2:3, :]
    ni = ni_ref[0]
    dn = dn_ref[0].astype(jnp.bfloat16).astype(jnp.float32)
    col_i = lax.broadcasted_iota(jnp.int32, (V, V), 1)
    acc = jnp.full(fs.shape, -1e30, jnp.float32)
    for k in range(20):
        oh = (ni[:, k:k + 1] == col_i).astype(jnp.float32)
        fsk = jnp.dot(oh, fs, preferred_element_type=jnp.float32, precision=lax.Precision.HIGHEST)
        th = jnp.maximum(dn[:, 3 * k:3 * k + 1] * s0
                         + dn[:, 3 * k + 1:3 * k + 2] * s1
                         + dn[:, 3 * k + 2:3 * k + 3] * s2, 0.0)
        acc = jnp.maximum(acc, th * fsk)
    out_ref[0] = fc_ref[0] + acc


def _conv(fm, ni, dn, p, oc):
    bs, V, ic = fm.shape
    wc, ws = p['w'][:, :oc], p['w'][:, oc:]
    bc, bsv = p['b'][:oc].reshape(1, oc), p['b'][oc:].reshape(1, oc)
    fc, fs = pl.pallas_call(
        _matmul_body,
        grid=(bs,),
        in_specs=[pl.BlockSpec((1, V, ic), lambda b: (b, 0, 0)),
                  pl.BlockSpec((ic, oc), lambda b: (0, 0)),
                  pl.BlockSpec((ic, oc), lambda b: (0, 0)),
                  pl.BlockSpec((1, oc), lambda b: (0, 0)),
                  pl.BlockSpec((1, oc), lambda b: (0, 0))],
        out_specs=(pl.BlockSpec((1, V, oc), lambda b: (b, 0, 0)),
                   pl.BlockSpec((1, V, oc), lambda b: (b, 0, 0))),
        out_shape=(jax.ShapeDtypeStruct((bs, V, oc), jnp.float32),
                   jax.ShapeDtypeStruct((bs, V, oc), jnp.float32)),
    )(fm, wc, ws, bc, bsv)
    return pl.pallas_call(
        _agg_body,
        grid=(bs,),
        in_specs=[pl.BlockSpec((1, V, oc), lambda b: (b, 0, 0)),
                  pl.BlockSpec((1, V, oc), lambda b: (b, 0, 0)),
                  pl.BlockSpec((1, V, 20), lambda b: (b, 0, 0)),
                  pl.BlockSpec((1, V, 60), lambda b: (b, 0, 0)),
                  pl.BlockSpec((3, oc), lambda b: (0, 0))],
        out_specs=pl.BlockSpec((1, V, oc), lambda b: (b, 0, 0)),
        out_shape=jax.ShapeDtypeStruct((bs, V, oc), jnp.float32),
    )(fc, fs, ni, dn, p['d'])


def _pool_body(fm_ref, ni_ref, out_ref):
    x = fm_ref[0]
    V = x.shape[0]
    ni = ni_ref[0]
    col_i = lax.broadcasted_iota(jnp.int32, (V, V), 1)
    acc = jnp.full(x.shape, -1e30, jnp.float32)
    for k in range(20):
        oh = (ni[:, k:k + 1] == col_i).astype(jnp.float32)
        acc = jnp.maximum(acc, jnp.dot(oh, x, preferred_element_type=jnp.float32, precision=lax.Precision.HIGHEST))
    out_ref[0] = acc


def _pool(fm, ni):
    bs, V, c = fm.shape
    return pl.pallas_call(
        _pool_body,
        grid=(bs,),
        in_specs=[pl.BlockSpec((1, V, c), lambda b: (b, 0, 0)),
                  pl.BlockSpec((1, V, 20), lambda b: (b, 0, 0))],
        out_specs=pl.BlockSpec((1, V, c), lambda b: (b, 0, 0)),
        out_shape=jax.ShapeDtypeStruct((bs, V, c), jnp.float32),
    )(fm, ni)


def _head_body(fm_ref, w1_ref, b1_ref, g_ref, bb_ref, w2_ref, b2_ref, out_ref):
    bs = fm_ref.shape[0]
    fg = jnp.concatenate(
        [jnp.max(fm_ref[b], axis=0, keepdims=True) for b in range(bs)], axis=0)
    h = jnp.dot(fg, w1_ref[...], preferred_element_type=jnp.float32) + b1_ref[...]
    mu = jnp.mean(h, axis=0, keepdims=True)
    va = jnp.mean((h - mu) ** 2, axis=0, keepdims=True)
    hn = (h - mu) / jnp.sqrt(va + 1e-5) * g_ref[...] + bb_ref[...]
    hr = jnp.maximum(hn, 0.0)
    out_ref[...] = jnp.dot(hr, w2_ref[...], preferred_element_type=jnp.float32) \
        + b2_ref[...]


def _head(fm, params):
    bs, V, c = fm.shape
    w1, w2 = params['fc1_w'], params['fc2_w']
    b1 = params['fc1_b'].reshape(1, -1)
    b2 = params['fc2_b'].reshape(1, -1)
    g = params['bn_g'].reshape(1, -1)
    bb = params['bn_b'].reshape(1, -1)
    n_out = w2.shape[1]
    return pl.pallas_call(
        _head_body,
        out_shape=jax.ShapeDtypeStruct((bs, n_out), jnp.float32),
    )(fm, w1, b1, g, bb, w2, b2)


def kernel(vertices, params):
    p = params
    ni1, dn1 = _knn(vertices)
    fm = _conv(vertices, ni1, dn1, p['conv_0'], 32)
    fm = _conv(fm, ni1, dn1, p['conv_1'], 64)
    fm = _conv(fm, ni1, dn1, p['conv_1_2'], 64)
    fm = _conv(fm, ni1, dn1, p['conv_1_3'], 64)
    fm = _conv(fm, ni1, dn1, p['conv_1_4'], 64)
    fm = _conv(fm, ni1, dn1, p['pool_1'], 64)
    fm = _pool(fm, ni1)
    v2 = vertices[:, ::2, :]
    fm = fm[:, ::2, :]
    ni2, dn2 = _knn(v2)
    fm = _conv(fm, ni2, dn2, p['conv_2'], 128)
    fm = _conv(fm, ni2, dn2, p['conv_3'], 256)
    fm = _conv(fm, ni2, dn2, p['conv_4'], 1024)
    fm = _conv(fm, ni2, dn2, p['conv_4_2'], 1024)
    fm = _conv(fm, ni2, dn2, p['conv_4_3'], 1024)
    fm = _conv(fm, ni2, dn2, p['conv_4_4'], 1024)
    return _head(fm, params)


# speedup vs baseline: 6.8667x; 1.8667x over previous
import jax
import jax.numpy as jnp
from jax import lax
from jax.experimental import pallas as pl


def _knn_body(verts_ref, ni_ref, dn_ref):
    x = verts_ref[0]
    V = x.shape[0]
    inner = lax.dot_general(x, x, (((1,), (1,)), ((), ())),
                            preferred_element_type=jnp.float32)
    qcol = jnp.sum(x * x, axis=1, keepdims=True)
    row_i = lax.broadcasted_iota(jnp.int32, (V, V), 0)
    col_i = lax.broadcasted_iota(jnp.int32, (V, V), 1)
    eye = row_i == col_i
    qrow = jnp.sum(jnp.where(eye, qcol, 0.0), axis=0, keepdims=True)
    D = -2.0 * inner + qcol + qrow
    xr = [jnp.sum(jnp.where(eye, x[:, c:c + 1], 0.0), axis=0, keepdims=True)
          for c in range(3)]
    for k in range(21):
        mv = jnp.min(D, axis=1, keepdims=True)
        idx = jnp.min(jnp.where(D == mv, col_i, V), axis=1, keepdims=True)
        m = col_i == idx
        D = jnp.where(m, 1e30, D)
        if k > 0:
            ni_ref[0, :, k - 1:k] = idx
            nv = [jnp.sum(jnp.where(m, xr[c], 0.0), axis=1, keepdims=True)
                  for c in range(3)]
            dv = [nv[c] - x[:, c:c + 1] for c in range(3)]
            nrm = jnp.sqrt(dv[0] * dv[0] + dv[1] * dv[1] + dv[2] * dv[2]) + 1e-12
            dn_ref[0, :, 3 * (k - 1):3 * k] = jnp.concatenate(
                [dv[c] / nrm for c in range(3)], axis=1)


def _knn(verts):
    bs, V, _ = verts.shape
    return pl.pallas_call(
        _knn_body,
        grid=(bs,),
        in_specs=[pl.BlockSpec((1, V, 3), lambda b: (b, 0, 0))],
        out_specs=(pl.BlockSpec((1, V, 20), lambda b: (b, 0, 0)),
                   pl.BlockSpec((1, V, 60), lambda b: (b, 0, 0))),
        out_shape=(jax.ShapeDtypeStruct((bs, V, 20), jnp.int32),
                   jax.ShapeDtypeStruct((bs, V, 60), jnp.float32)),
    )(verts)


def _matmul_body(fm_ref, wc_ref, ws_ref, bc_ref, bs_ref, fc_ref, fs_ref):
    x = fm_ref[0]
    fc_ref[0] = jnp.dot(x, wc_ref[...],
                        preferred_element_type=jnp.float32) + bc_ref[...]
    fs_ref[0] = jnp.dot(x, ws_ref[...],
                        preferred_element_type=jnp.float32) + bs_ref[...]


def _agg_body(fc_ref, fs_ref, ni_ref, dn_ref, d_ref, out_ref):
    fs = fs_ref[0]
    V = fs.shape[0]
    d = d_ref[...]
    nrm = jnp.sqrt(jnp.sum(d * d, axis=0, keepdims=True))
    s = d / (nrm + 1e-12)
    ni = ni_ref[0]
    dn = dn_ref[0]
    col_i = lax.broadcasted_iota(jnp.int32, (V, V), 1)
    fs_hi = fs.astype(jnp.bfloat16)
    r1 = fs - fs_hi.astype(jnp.float32)
    fs_mid = r1.astype(jnp.bfloat16)
    fs_lo = (r1 - fs_mid.astype(jnp.float32)).astype(jnp.bfloat16)
    acc = jnp.full(fs.shape, -1e30, jnp.float32)
    for k in range(20):
        oh = (ni[:, k:k + 1] == col_i).astype(jnp.bfloat16)
        fsk = (jnp.dot(oh, fs_hi, preferred_element_type=jnp.float32)
               + jnp.dot(oh, fs_mid, preferred_element_type=jnp.float32)
               + jnp.dot(oh, fs_lo, preferred_element_type=jnp.float32))
        th = jnp.maximum(jnp.dot(dn[:, 3 * k:3 * k + 3], s,
                                 preferred_element_type=jnp.float32), 0.0)
        acc = jnp.maximum(acc, th * fsk)
    out_ref[0] = fc_ref[0] + acc


def _conv(fm, ni, dn, p, oc):
    bs, V, ic = fm.shape
    wc, ws = p['w'][:, :oc], p['w'][:, oc:]
    bc, bsv = p['b'][:oc].reshape(1, oc), p['b'][oc:].reshape(1, oc)
    fc, fs = pl.pallas_call(
        _matmul_body,
        grid=(bs,),
        in_specs=[pl.BlockSpec((1, V, ic), lambda b: (b, 0, 0)),
                  pl.BlockSpec((ic, oc), lambda b: (0, 0)),
                  pl.BlockSpec((ic, oc), lambda b: (0, 0)),
                  pl.BlockSpec((1, oc), lambda b: (0, 0)),
                  pl.BlockSpec((1, oc), lambda b: (0, 0))],
        out_specs=(pl.BlockSpec((1, V, oc), lambda b: (b, 0, 0)),
                   pl.BlockSpec((1, V, oc), lambda b: (b, 0, 0))),
        out_shape=(jax.ShapeDtypeStruct((bs, V, oc), jnp.float32),
                   jax.ShapeDtypeStruct((bs, V, oc), jnp.float32)),
    )(fm, wc, ws, bc, bsv)
    return pl.pallas_call(
        _agg_body,
        grid=(bs,),
        in_specs=[pl.BlockSpec((1, V, oc), lambda b: (b, 0, 0)),
                  pl.BlockSpec((1, V, oc), lambda b: (b, 0, 0)),
                  pl.BlockSpec((1, V, 20), lambda b: (b, 0, 0)),
                  pl.BlockSpec((1, V, 60), lambda b: (b, 0, 0)),
                  pl.BlockSpec((3, oc), lambda b: (0, 0))],
        out_specs=pl.BlockSpec((1, V, oc), lambda b: (b, 0, 0)),
        out_shape=jax.ShapeDtypeStruct((bs, V, oc), jnp.float32),
    )(fc, fs, ni, dn, p['d'])


def _pool_body(fm_ref, ni_ref, out_ref):
    x = fm_ref[0]
    V = x.shape[0]
    ni = ni_ref[0]
    col_i = lax.broadcasted_iota(jnp.int32, (V, V), 1)
    x_hi = x.astype(jnp.bfloat16)
    r1 = x - x_hi.astype(jnp.float32)
    x_mid = r1.astype(jnp.bfloat16)
    x_lo = (r1 - x_mid.astype(jnp.float32)).astype(jnp.bfloat16)
    acc = jnp.full(x.shape, -1e30, jnp.float32)
    for k in range(20):
        oh = (ni[:, k:k + 1] == col_i).astype(jnp.bfloat16)
        acc = jnp.maximum(
            acc, jnp.dot(oh, x_hi, preferred_element_type=jnp.float32)
            + jnp.dot(oh, x_mid, preferred_element_type=jnp.float32)
            + jnp.dot(oh, x_lo, preferred_element_type=jnp.float32))
    out_ref[0] = acc


def _pool(fm, ni):
    bs, V, c = fm.shape
    return pl.pallas_call(
        _pool_body,
        grid=(bs,),
        in_specs=[pl.BlockSpec((1, V, c), lambda b: (b, 0, 0)),
                  pl.BlockSpec((1, V, 20), lambda b: (b, 0, 0))],
        out_specs=pl.BlockSpec((1, V, c), lambda b: (b, 0, 0)),
        out_shape=jax.ShapeDtypeStruct((bs, V, c), jnp.float32),
    )(fm, ni)


def _head_body(fm_ref, w1_ref, b1_ref, g_ref, bb_ref, w2_ref, b2_ref, out_ref):
    bs = fm_ref.shape[0]
    fg = jnp.concatenate(
        [jnp.max(fm_ref[b], axis=0, keepdims=True) for b in range(bs)], axis=0)
    h = jnp.dot(fg, w1_ref[...], preferred_element_type=jnp.float32) + b1_ref[...]
    mu = jnp.mean(h, axis=0, keepdims=True)
    va = jnp.mean((h - mu) ** 2, axis=0, keepdims=True)
    hn = (h - mu) / jnp.sqrt(va + 1e-5) * g_ref[...] + bb_ref[...]
    hr = jnp.maximum(hn, 0.0)
    out_ref[...] = jnp.dot(hr, w2_ref[...], preferred_element_type=jnp.float32) \
        + b2_ref[...]


def _head(fm, params):
    bs, V, c = fm.shape
    w1, w2 = params['fc1_w'], params['fc2_w']
    b1 = params['fc1_b'].reshape(1, -1)
    b2 = params['fc2_b'].reshape(1, -1)
    g = params['bn_g'].reshape(1, -1)
    bb = params['bn_b'].reshape(1, -1)
    n_out = w2.shape[1]
    return pl.pallas_call(
        _head_body,
        out_shape=jax.ShapeDtypeStruct((bs, n_out), jnp.float32),
    )(fm, w1, b1, g, bb, w2, b2)


def kernel(vertices, params):
    p = params
    ni1, dn1 = _knn(vertices)
    fm = _conv(vertices, ni1, dn1, p['conv_0'], 32)
    fm = _conv(fm, ni1, dn1, p['conv_1'], 64)
    fm = _conv(fm, ni1, dn1, p['conv_1_2'], 64)
    fm = _conv(fm, ni1, dn1, p['conv_1_3'], 64)
    fm = _conv(fm, ni1, dn1, p['conv_1_4'], 64)
    fm = _conv(fm, ni1, dn1, p['pool_1'], 64)
    fm = _pool(fm, ni1)
    v2 = vertices[:, ::2, :]
    fm = fm[:, ::2, :]
    ni2, dn2 = _knn(v2)
    fm = _conv(fm, ni2, dn2, p['conv_2'], 128)
    fm = _conv(fm, ni2, dn2, p['conv_3'], 256)
    fm = _conv(fm, ni2, dn2, p['conv_4'], 1024)
    fm = _conv(fm, ni2, dn2, p['conv_4_2'], 1024)
    fm = _conv(fm, ni2, dn2, p['conv_4_3'], 1024)
    fm = _conv(fm, ni2, dn2, p['conv_4_4'], 1024)
    return _head(fm, params)


# speedup vs baseline: 8.8426x; 1.2878x over previous
import functools

import jax
import jax.numpy as jnp
from jax import lax
from jax.experimental import pallas as pl
from jax.experimental.pallas import tpu as pltpu
from jax.experimental.pallas import tpu_sc as plsc


def _knn_body(verts_ref, ni_ref, dn_ref):
    x = verts_ref[0]
    V = x.shape[0]
    inner = lax.dot_general(x, x, (((1,), (1,)), ((), ())),
                            preferred_element_type=jnp.float32)
    qcol = jnp.sum(x * x, axis=1, keepdims=True)
    row_i = lax.broadcasted_iota(jnp.int32, (V, V), 0)
    col_i = lax.broadcasted_iota(jnp.int32, (V, V), 1)
    eye = row_i == col_i
    qrow = jnp.sum(jnp.where(eye, qcol, 0.0), axis=0, keepdims=True)
    D = -2.0 * inner + qcol + qrow
    xr = [jnp.sum(jnp.where(eye, x[:, c:c + 1], 0.0), axis=0, keepdims=True)
          for c in range(3)]
    for k in range(21):
        mv = jnp.min(D, axis=1, keepdims=True)
        idx = jnp.min(jnp.where(D == mv, col_i, V), axis=1, keepdims=True)
        m = col_i == idx
        D = jnp.where(m, 1e30, D)
        if k > 0:
            ni_ref[0, :, k - 1:k] = idx
            nv = [jnp.sum(jnp.where(m, xr[c], 0.0), axis=1, keepdims=True)
                  for c in range(3)]
            dv = [nv[c] - x[:, c:c + 1] for c in range(3)]
            nrm = jnp.sqrt(dv[0] * dv[0] + dv[1] * dv[1] + dv[2] * dv[2]) + 1e-12
            dn_ref[0, :, 3 * (k - 1):3 * k] = jnp.concatenate(
                [dv[c] / nrm for c in range(3)], axis=1)


def _knn(verts):
    bs, V, _ = verts.shape
    return pl.pallas_call(
        _knn_body,
        grid=(bs,),
        in_specs=[pl.BlockSpec((1, V, 3), lambda b: (b, 0, 0))],
        out_specs=(pl.BlockSpec((1, V, 20), lambda b: (b, 0, 0)),
                   pl.BlockSpec((1, V, 60), lambda b: (b, 0, 0))),
        out_shape=(jax.ShapeDtypeStruct((bs, V, 20), jnp.int32),
                   jax.ShapeDtypeStruct((bs, V, 60), jnp.float32)),
    )(verts)


def _matmul_body(fm_ref, wc_ref, ws_ref, bc_ref, bs_ref, fc_ref, fs_ref):
    x = fm_ref[0]
    fc_ref[0] = jnp.dot(x, wc_ref[...],
                        preferred_element_type=jnp.float32) + bc_ref[...]
    fs_ref[0] = jnp.dot(x, ws_ref[...],
                        preferred_element_type=jnp.float32) + bs_ref[...]


def _agg_body(fc_ref, fs_ref, ni_ref, dn_ref, d_ref, out_ref):
    fs = fs_ref[0]
    V = fs.shape[0]
    d = d_ref[...]
    nrm = jnp.sqrt(jnp.sum(d * d, axis=0, keepdims=True))
    s = d / (nrm + 1e-12)
    ni = ni_ref[0]
    dn = dn_ref[0]
    col_i = lax.broadcasted_iota(jnp.int32, (V, V), 1)
    fs_hi = fs.astype(jnp.bfloat16)
    r1 = fs - fs_hi.astype(jnp.float32)
    fs_mid = r1.astype(jnp.bfloat16)
    fs_lo = (r1 - fs_mid.astype(jnp.float32)).astype(jnp.bfloat16)
    acc = jnp.full(fs.shape, -1e30, jnp.float32)
    for k in range(20):
        oh = (ni[:, k:k + 1] == col_i).astype(jnp.bfloat16)
        fsk = (jnp.dot(oh, fs_hi, preferred_element_type=jnp.float32)
               + jnp.dot(oh, fs_mid, preferred_element_type=jnp.float32)
               + jnp.dot(oh, fs_lo, preferred_element_type=jnp.float32))
        th = jnp.maximum(jnp.dot(dn[:, 3 * k:3 * k + 3], s,
                                 preferred_element_type=jnp.float32), 0.0)
        acc = jnp.maximum(acc, th * fsk)
    out_ref[0] = fc_ref[0] + acc


def _conv(fm, ni, dn, p, oc):
    bs, V, ic = fm.shape
    wc, ws = p['w'][:, :oc], p['w'][:, oc:]
    bc, bsv = p['b'][:oc].reshape(1, oc), p['b'][oc:].reshape(1, oc)
    fc, fs = pl.pallas_call(
        _matmul_body,
        grid=(bs,),
        in_specs=[pl.BlockSpec((1, V, ic), lambda b: (b, 0, 0)),
                  pl.BlockSpec((ic, oc), lambda b: (0, 0)),
                  pl.BlockSpec((ic, oc), lambda b: (0, 0)),
                  pl.BlockSpec((1, oc), lambda b: (0, 0)),
                  pl.BlockSpec((1, oc), lambda b: (0, 0))],
        out_specs=(pl.BlockSpec((1, V, oc), lambda b: (b, 0, 0)),
                   pl.BlockSpec((1, V, oc), lambda b: (b, 0, 0))),
        out_shape=(jax.ShapeDtypeStruct((bs, V, oc), jnp.float32),
                   jax.ShapeDtypeStruct((bs, V, oc), jnp.float32)),
    )(fm, wc, ws, bc, bsv)
    return pl.pallas_call(
        _agg_body,
        grid=(bs,),
        in_specs=[pl.BlockSpec((1, V, oc), lambda b: (b, 0, 0)),
                  pl.BlockSpec((1, V, oc), lambda b: (b, 0, 0)),
                  pl.BlockSpec((1, V, 20), lambda b: (b, 0, 0)),
                  pl.BlockSpec((1, V, 60), lambda b: (b, 0, 0)),
                  pl.BlockSpec((3, oc), lambda b: (0, 0))],
        out_specs=pl.BlockSpec((1, V, oc), lambda b: (b, 0, 0)),
        out_shape=jax.ShapeDtypeStruct((bs, V, oc), jnp.float32),
    )(fc, fs, ni, dn, p['d'])


def _sc_gather(table, idx, chunk):
    B = idx.shape[0]
    D = table.shape[1]
    NW = 32
    b_per_w = B // NW
    n_chunks = b_per_w // chunk

    @functools.partial(
        pl.kernel,
        mesh=plsc.VectorSubcoreMesh(core_axis_name="c", subcore_axis_name="s"),
        out_type=jax.ShapeDtypeStruct((B, D), jnp.float32),
        scratch_types=[
            pltpu.VMEM((chunk,), jnp.int32),
            pltpu.VMEM((chunk, D), jnp.float32),
            pltpu.SemaphoreType.DMA,
        ],
    )
    def k(table_hbm, idx_hbm, out_hbm, idx_v, rows_v, sem):
        wid = lax.axis_index("s") * 2 + lax.axis_index("c")
        base = wid * b_per_w

        def body(i, carry):
            off = base + i * chunk
            pltpu.sync_copy(idx_hbm.at[pl.ds(off, chunk)], idx_v)
            pltpu.async_copy(table_hbm.at[idx_v], rows_v, sem).wait()
            pltpu.sync_copy(rows_v, out_hbm.at[pl.ds(off, chunk)])
            return carry

        lax.fori_loop(0, n_chunks, body, 0)

    return k(table, idx)


def _agg_sc_body(fc_ref, g_ref, dn_ref, d_ref, out_ref):
    fc = fc_ref[0]
    V = fc.shape[0]
    d = d_ref[...]
    nrm = jnp.sqrt(jnp.sum(d * d, axis=0, keepdims=True))
    s = d / (nrm + 1e-12)
    dn = dn_ref[0]
    acc = jnp.full(fc.shape, -1e30, jnp.float32)
    oc = fc.shape[1]
    for k in range(20):
        fsk = g_ref[0, k * V:(k + 1) * V, :oc]
        th = jnp.maximum(jnp.dot(dn[:, 3 * k:3 * k + 3], s,
                                 preferred_element_type=jnp.float32), 0.0)
        acc = jnp.maximum(acc, th * fsk)
    out_ref[0] = fc + acc


def _matmul(fm, p, oc, dp=None):
    bs, V, ic = fm.shape
    dp = dp or oc
    wc, ws = p['w'][:, :oc], p['w'][:, oc:]
    bc, bsv = p['b'][:oc].reshape(1, oc), p['b'][oc:].reshape(1, oc)
    if dp != oc:
        ws = jnp.pad(ws, ((0, 0), (0, dp - oc)))
        bsv = jnp.pad(bsv, ((0, 0), (0, dp - oc)))
    return pl.pallas_call(
        _matmul_body,
        grid=(bs,),
        in_specs=[pl.BlockSpec((1, V, ic), lambda b: (b, 0, 0)),
                  pl.BlockSpec((ic, oc), lambda b: (0, 0)),
                  pl.BlockSpec((ic, dp), lambda b: (0, 0)),
                  pl.BlockSpec((1, oc), lambda b: (0, 0)),
                  pl.BlockSpec((1, dp), lambda b: (0, 0))],
        out_specs=(pl.BlockSpec((1, V, oc), lambda b: (b, 0, 0)),
                   pl.BlockSpec((1, V, dp), lambda b: (b, 0, 0))),
        out_shape=(jax.ShapeDtypeStruct((bs, V, oc), jnp.float32),
                   jax.ShapeDtypeStruct((bs, V, dp), jnp.float32)),
    )(fm, wc, ws, bc, bsv)


def _conv_sc(fm, gidx, dn, p, oc, chunk):
    bs, V, ic = fm.shape
    dp = max(oc, 128)
    fc, fs = _matmul(fm, p, oc, dp)
    g = _sc_gather(fs.reshape(bs * V, dp), gidx, chunk)
    g = g.reshape(bs, 20 * V, dp)
    return pl.pallas_call(
        _agg_sc_body,
        grid=(bs,),
        in_specs=[pl.BlockSpec((1, V, oc), lambda b: (b, 0, 0)),
                  pl.BlockSpec((1, 20 * V, dp), lambda b: (b, 0, 0)),
                  pl.BlockSpec((1, V, 60), lambda b: (b, 0, 0)),
                  pl.BlockSpec((3, oc), lambda b: (0, 0))],
        out_specs=pl.BlockSpec((1, V, oc), lambda b: (b, 0, 0)),
        out_shape=jax.ShapeDtypeStruct((bs, V, oc), jnp.float32),
    )(fc, g, dn, p['d'])


def _pool_max_body(g_ref, out_ref):
    V = out_ref.shape[1]
    c = out_ref.shape[2]
    acc = jnp.full(out_ref.shape[1:], -1e30, jnp.float32)
    for k in range(20):
        acc = jnp.maximum(acc, g_ref[0, k * V:(k + 1) * V, :c])
    out_ref[0] = acc


def _pool_sc(fm, gidx, chunk):
    bs, V, c = fm.shape
    dp = max(c, 128)
    fm2 = fm.reshape(bs * V, c)
    if dp != c:
        fm2 = jnp.pad(fm2, ((0, 0), (0, dp - c)))
    g = _sc_gather(fm2, gidx, chunk)
    g = g.reshape(bs, 20 * V, dp)
    return pl.pallas_call(
        _pool_max_body,
        grid=(bs,),
        in_specs=[pl.BlockSpec((1, 20 * V, dp), lambda b: (b, 0, 0))],
        out_specs=pl.BlockSpec((1, V, c), lambda b: (b, 0, 0)),
        out_shape=jax.ShapeDtypeStruct((bs, V, c), jnp.float32),
    )(g)


def _pool_body(fm_ref, ni_ref, out_ref):
    x = fm_ref[0]
    V = x.shape[0]
    ni = ni_ref[0]
    col_i = lax.broadcasted_iota(jnp.int32, (V, V), 1)
    x_hi = x.astype(jnp.bfloat16)
    r1 = x - x_hi.astype(jnp.float32)
    x_mid = r1.astype(jnp.bfloat16)
    x_lo = (r1 - x_mid.astype(jnp.float32)).astype(jnp.bfloat16)
    acc = jnp.full(x.shape, -1e30, jnp.float32)
    for k in range(20):
        oh = (ni[:, k:k + 1] == col_i).astype(jnp.bfloat16)
        acc = jnp.maximum(
            acc, jnp.dot(oh, x_hi, preferred_element_type=jnp.float32)
            + jnp.dot(oh, x_mid, preferred_element_type=jnp.float32)
            + jnp.dot(oh, x_lo, preferred_element_type=jnp.float32))
    out_ref[0] = acc


def _pool(fm, ni):
    bs, V, c = fm.shape
    return pl.pallas_call(
        _pool_body,
        grid=(bs,),
        in_specs=[pl.BlockSpec((1, V, c), lambda b: (b, 0, 0)),
                  pl.BlockSpec((1, V, 20), lambda b: (b, 0, 0))],
        out_specs=pl.BlockSpec((1, V, c), lambda b: (b, 0, 0)),
        out_shape=jax.ShapeDtypeStruct((bs, V, c), jnp.float32),
    )(fm, ni)


def _head_body(fm_ref, w1_ref, b1_ref, g_ref, bb_ref, w2_ref, b2_ref, out_ref):
    bs = fm_ref.shape[0]
    fg = jnp.concatenate(
        [jnp.max(fm_ref[b], axis=0, keepdims=True) for b in range(bs)], axis=0)
    h = jnp.dot(fg, w1_ref[...], preferred_element_type=jnp.float32) + b1_ref[...]
    mu = jnp.mean(h, axis=0, keepdims=True)
    va = jnp.mean((h - mu) ** 2, axis=0, keepdims=True)
    hn = (h - mu) / jnp.sqrt(va + 1e-5) * g_ref[...] + bb_ref[...]
    hr = jnp.maximum(hn, 0.0)
    out_ref[...] = jnp.dot(hr, w2_ref[...], preferred_element_type=jnp.float32) \
        + b2_ref[...]


def _head(fm, params):
    bs, V, c = fm.shape
    w1, w2 = params['fc1_w'], params['fc2_w']
    b1 = params['fc1_b'].reshape(1, -1)
    b2 = params['fc2_b'].reshape(1, -1)
    g = params['bn_g'].reshape(1, -1)
    bb = params['bn_b'].reshape(1, -1)
    n_out = w2.shape[1]
    return pl.pallas_call(
        _head_body,
        out_shape=jax.ShapeDtypeStruct((bs, n_out), jnp.float32),
    )(fm, w1, b1, g, bb, w2, b2)


def kernel(vertices, params):
    p = params
    bs, V1, _ = vertices.shape
    ni1, dn1 = _knn(vertices)
    gidx1 = (jnp.arange(bs, dtype=jnp.int32)[:, None, None] * V1
             + jnp.transpose(ni1, (0, 2, 1))).reshape(-1)
    fm = _conv_sc(vertices, gidx1, dn1, p['conv_0'], 32, 128)
    fm = _conv_sc(fm, gidx1, dn1, p['conv_1'], 64, 128)
    fm = _conv_sc(fm, gidx1, dn1, p['conv_1_2'], 64, 128)
    fm = _conv_sc(fm, gidx1, dn1, p['conv_1_3'], 64, 128)
    fm = _conv_sc(fm, gidx1, dn1, p['conv_1_4'], 64, 128)
    fm = _conv_sc(fm, gidx1, dn1, p['pool_1'], 64, 128)
    fm = _pool_sc(fm, gidx1, 128)
    v2 = vertices[:, ::2, :]
    fm = fm[:, ::2, :]
    ni2, dn2 = _knn(v2)
    fm = _conv(fm, ni2, dn2, p['conv_2'], 128)
    fm = _conv(fm, ni2, dn2, p['conv_3'], 256)
    fm = _conv(fm, ni2, dn2, p['conv_4'], 1024)
    fm = _conv(fm, ni2, dn2, p['conv_4_2'], 1024)
    fm = _conv(fm, ni2, dn2, p['conv_4_3'], 1024)
    fm = _conv(fm, ni2, dn2, p['conv_4_4'], 1024)
    return _head(fm, params)
